# baseline (speedup 1.0000x reference)
import functools

import jax
import jax.numpy as jnp
from jax.experimental import pallas as pl
from jax.experimental.pallas import tpu as pltpu

_N0, _N1, _N2 = 10000, 15000, 22500
_D = 128
_K = 2


def _mm_body(x_ref, g1_ref, g2_ref, norm_ref, w_ref, b_ref,
             out_ref, *, relu):
    nrm = norm_ref[...]
    h0 = x_ref[...]
    h1 = g1_ref[...] * nrm
    h2 = g2_ref[...] * nrm
    cat = jnp.concatenate([h0, h1, h2], axis=1)
    acc = jnp.dot(cat, w_ref[...], preferred_element_type=jnp.float32)
    acc = acc + b_ref[...]
    if relu:
        acc = jnp.maximum(acc, 0.0)
    out_ref[...] = acc


def _mm_layer(x, g1, g2, norm2d, w, b, *, relu):
    np_rows = x.shape[0]
    tr = 512
    grid = (np_rows // tr,)
    bs_rows = pl.BlockSpec((tr, _D), lambda i: (i, 0))
    bs_norm = pl.BlockSpec((tr, 1), lambda i: (i, 0))
    bs_w = pl.BlockSpec((3 * _D, _D), lambda i: (0, 0))
    bs_b = pl.BlockSpec((1, _D), lambda i: (0, 0))
    fn = pl.pallas_call(
        functools.partial(_mm_body, relu=relu),
        grid=grid,
        in_specs=[bs_rows, bs_rows, bs_rows, bs_norm, bs_w, bs_b],
        out_specs=bs_rows,
        out_shape=jax.ShapeDtypeStruct((np_rows, _D), jnp.float32),
        compiler_params=pltpu.CompilerParams(
            dimension_semantics=("arbitrary",)),
    )
    return fn(x, g1, g2, norm2d, w, b.reshape(1, _D))


def _sym_edges(ei):
    src = jnp.concatenate([ei[0], ei[1]])
    dst = jnp.concatenate([ei[1], ei[0]])
    return src, dst


def _segsum(t, src, dst, n):
    return jax.ops.segment_sum(t[src], dst, num_segments=n)


def _ref_block_equiv(h, t1, Ws, bs, src, dst, norm2d, normsq2d, npad):
    eltwise = (2, 4, 6, 8, 10, 12)
    hidden = []
    for i in range(14):
        relu = i < 13
        g1 = _segsum(t1, src, dst, npad)
        t2 = (g1 * norm2d) * norm2d
        g2 = _segsum(t2, src, dst, npad)
        out = _mm_layer(h, g1, g2, norm2d, Ws[i], bs[i], relu=relu)
        hidden.append(out)
        h = out
        if i in eltwise:
            h = 0.5 * (hidden[-2] + h)
        t1 = h * norm2d
    return h


def _unpool(feat, pool_idx, n_old, n_new_pad):
    new_vs = 0.5 * jnp.sum(feat[pool_idx], axis=1)
    out = jnp.zeros((n_new_pad, _D), jnp.float32)
    out = out.at[:n_old].set(feat[:n_old])
    out = out.at[n_old:n_old + new_vs.shape[0]].set(new_vs)
    return out


def _norms(dst, n, npad):
    deg = jnp.zeros((npad,), jnp.float32).at[dst].add(1.0)
    norm = jnp.power(jnp.clip(deg, 1.0, None), -0.5)
    return norm.reshape(npad, 1), (norm * norm).reshape(npad, 1)


def _pad_rows(x, npad):
    return jnp.pad(x, ((0, npad - x.shape[0]), (0, 0)))


def kernel(features, edge_index0, pool_idx0, edge_index1, pool_idx1,
           edge_index2, W, b):
    np0, np1, np2 = 10240, 15104, 22528

    src0, dst0 = _sym_edges(edge_index0)
    src1, dst1 = _sym_edges(edge_index1)
    src2, dst2 = _sym_edges(edge_index2)

    n0_2d, nsq0 = _norms(dst0, _N0, np0)
    n1_2d, nsq1 = _norms(dst1, _N1, np1)
    n2_2d, nsq2 = _norms(dst2, _N2, np2)

    h = _pad_rows(features, np0)
    t1 = h * n0_2d
    out1 = _ref_block_equiv(h, t1, W[0:14], b[0:14], src0, dst0, n0_2d, nsq0, np0)

    h1 = _unpool(out1, pool_idx0, _N0, np1)
    t1 = h1 * n1_2d
    out2 = _ref_block_equiv(h1, t1, W[14:28], b[14:28], src1, dst1, n1_2d, nsq1, np1)

    h2 = _unpool(out2, pool_idx1, _N1, np2)
    t1 = h2 * n2_2d
    out3 = _ref_block_equiv(h2, t1, W[28:42], b[28:42], src2, dst2, n2_2d, nsq2, np2)

    return (out1[:_N0], out2[:_N1], out3[:_N2], h1[:_N1], h2[:_N2])


# speedup vs baseline: 4.9923x; 4.9923x over previous
import functools

import jax
import jax.numpy as jnp
from jax import lax
from jax.experimental import pallas as pl
from jax.experimental.pallas import tpu as pltpu
from jax.experimental.pallas import tpu_sc as plsc

_N0, _N1, _N2 = 10000, 15000, 22500
_D = 128

_NP = {0: 10240, 1: 15360, 2: 23040}
_NH = {b: _NP[b] // 2 for b in _NP}
_CH = 1024
_EP = {0: 327680, 1: 491520, 2: 737280}
_SCAN = {b: _EP[b] // 16 for b in _EP}
_G = {b: _SCAN[b] // 128 for b in _SCAN}
_GARB = 32


def _mesh():
    return plsc.VectorSubcoreMesh(core_axis_name="c", subcore_axis_name="s")


def _partition_body(src_hbm, dst_hbm,
                    srcp_hbm, dstp_hbm, counts_hbm, deg_hbm,
                    sv, dv, srcp_v, dstp_v, onev, cntv, z1, deg_sp, sem,
                    *, nh, scan, g):
    c = lax.axis_index("c")
    s = lax.axis_index("s")
    lo = c * nh
    iota = lax.iota(jnp.int32, 16)

    @pl.when(s == 0)
    def _zero_deg():
        def zfill(i, _):
            z1[pl.ds(i * 16, 16)] = jnp.zeros((16,), jnp.float32)
            return 0
        lax.fori_loop(0, 64, zfill, 0)
        full, rem = (nh + _GARB) // 1024, (nh + _GARB) % 1024
        for kk in range(full):
            pltpu.sync_copy(z1, deg_sp.at[pl.ds(kk * 1024, 1024)])
        if rem:
            pltpu.sync_copy(z1.at[pl.ds(0, rem)],
                            deg_sp.at[pl.ds(full * 1024, rem)])

    for i in range(8):
        onev[pl.ds(i * 16, 16)] = jnp.ones((16,), jnp.float32)

    base = s * scan

    def chunk_body(k, off):
        pltpu.sync_copy(src_hbm.at[pl.ds(base + k * _CH, _CH)], sv)
        pltpu.sync_copy(dst_hbm.at[pl.ds(base + k * _CH, _CH)], dv)

        def vec_body(j, off):
            s16 = sv[pl.ds(j * 16, 16)]
            d16 = dv[pl.ds(j * 16, 16)]
            m = (d16 >= lo) & (d16 < lo + nh)
            inc = jnp.where(m, 1, 0)
            pos = off + plsc.cumsum(inc) - 1
            row = lax.shift_right_logical(pos, 7)
            col = jnp.bitwise_and(pos, 127)
            plsc.store_scatter(srcp_v, [row, col], s16, mask=m)
            plsc.store_scatter(dstp_v, [row, col], d16 - lo, mask=m)
            return off + jnp.max(plsc.all_reduce_population_count(m))

        return lax.fori_loop(0, _CH // 16, vec_body, off)

    off = lax.fori_loop(0, scan // _CH, chunk_body, 0)

    target = ((off + 127) // 128) * 128

    def pad_cond(o):
        return o < target

    def pad_body(o):
        lanes = o + iota
        m = lanes < target
        row = lax.shift_right_logical(lanes, 7)
        col = jnp.bitwise_and(lanes, 127)
        plsc.store_scatter(srcp_v, [row, col], iota + s * 16, mask=m)
        plsc.store_scatter(dstp_v, [row, col],
                           jnp.full((16,), nh + s, jnp.int32), mask=m)
        return o + jnp.max(plsc.all_reduce_population_count(m))

    target = lax.while_loop(pad_cond, pad_body, off)

    cntv[...] = jnp.full((16,), target, jnp.int32)
    pltpu.sync_copy(cntv, counts_hbm.at[c, s])
    pltpu.sync_copy(srcp_v, srcp_hbm.at[c, s])
    pltpu.sync_copy(dstp_v, dstp_hbm.at[c, s])

    plsc.subcore_barrier()

    def deg_body(gi, _):
        pltpu.sync_copy(onev, deg_sp.at[dstp_v.at[gi]], add=True)
        return 0

    lax.fori_loop(0, target // 128, deg_body, 0)
    plsc.subcore_barrier()

    @pl.when(s == 0)
    def _drain_deg():
        full, rem = nh // 1024, nh % 1024
        for kk in range(full):
            pltpu.sync_copy(deg_sp.at[pl.ds(kk * 1024, 1024)], z1)
            pltpu.sync_copy(z1, deg_hbm.at[pl.ds(c * nh + kk * 1024, 1024)])
        if rem:
            pltpu.sync_copy(deg_sp.at[pl.ds(full * 1024, rem)],
                            z1.at[pl.ds(0, rem)])
            pltpu.sync_copy(z1.at[pl.ds(0, rem)],
                            deg_hbm.at[pl.ds(c * nh + full * 1024, rem)])


@functools.lru_cache(maxsize=None)
def _partition_fn(block):
    nh, scan, g = _NH[block], _SCAN[block], _G[block]
    npad = _NP[block]
    return pl.kernel(
        functools.partial(_partition_body, nh=nh, scan=scan, g=g),
        out_type=[
            jax.ShapeDtypeStruct((2, 16, g, 128), jnp.int32),
            jax.ShapeDtypeStruct((2, 16, g, 128), jnp.int32),
            jax.ShapeDtypeStruct((2, 16, 16), jnp.int32),
            jax.ShapeDtypeStruct((npad,), jnp.float32),
        ],
        mesh=_mesh(),
        scratch_types=[
            pltpu.VMEM((_CH,), jnp.int32),
            pltpu.VMEM((_CH,), jnp.int32),
            pltpu.VMEM((g, 128), jnp.int32),
            pltpu.VMEM((g, 128), jnp.int32),
            pltpu.VMEM((128,), jnp.float32),
            pltpu.VMEM((16,), jnp.int32),
            pltpu.VMEM((1024,), jnp.float32),
            pltpu.VMEM_SHARED((nh + _GARB,), jnp.float32),
            pltpu.SemaphoreType.DMA,
        ],
        compiler_params=pltpu.CompilerParams(needs_layout_passes=False),
        name=f"sc_partition_b{block}",
    )


def _hop_body(t_hbm, srcp_hbm, dstp_hbm, counts_hbm,
              g_hbm,
              sv, dv, rows_v, cntv, acc_sp, sem,
              *, nh):
    c = lax.axis_index("c")
    s = lax.axis_index("s")
    zr = (nh + _GARB) // 16
    dr = nh // 16

    def zfill(i, _):
        for kk in range(_D // 16):
            rows_v[i, pl.ds(kk * 16, 16)] = jnp.zeros((16,), jnp.float32)
        return 0
    lax.fori_loop(0, 128, zfill, 0)
    zfull, zrem = zr // 128, zr % 128
    for kk in range(zfull):
        pltpu.sync_copy(rows_v, acc_sp.at[pl.ds(s * zr + kk * 128, 128)])
    if zrem:
        pltpu.sync_copy(rows_v.at[pl.ds(0, zrem)],
                        acc_sp.at[pl.ds(s * zr + zfull * 128, zrem)])
    plsc.subcore_barrier()

    pltpu.sync_copy(counts_hbm.at[c, s], cntv)
    cnt = jnp.max(cntv[...])

    def grp(gi, _):
        pltpu.sync_copy(srcp_hbm.at[c, s, gi], sv)
        pltpu.sync_copy(dstp_hbm.at[c, s, gi], dv)
        pltpu.async_copy(t_hbm.at[sv], rows_v, sem).wait()
        pltpu.sync_copy(rows_v, acc_sp.at[dv], add=True)
        return 0

    lax.fori_loop(0, cnt // 128, grp, 0)
    plsc.subcore_barrier()

    dfull, drem = dr // 128, dr % 128
    for kk in range(dfull):
        pltpu.sync_copy(acc_sp.at[pl.ds(s * dr + kk * 128, 128)], rows_v)
        pltpu.sync_copy(rows_v,
                        g_hbm.at[pl.ds(c * nh + s * dr + kk * 128, 128)])
    if drem:
        pltpu.sync_copy(acc_sp.at[pl.ds(s * dr + dfull * 128, drem)],
                        rows_v.at[pl.ds(0, drem)])
        pltpu.sync_copy(rows_v.at[pl.ds(0, drem)],
                        g_hbm.at[pl.ds(c * nh + s * dr + dfull * 128, drem)])


@functools.lru_cache(maxsize=None)
def _hop_fn(block):
    nh, g = _NH[block], _G[block]
    npad = _NP[block]
    return pl.kernel(
        functools.partial(_hop_body, nh=nh),
        out_type=jax.ShapeDtypeStruct((npad, _D), jnp.float32),
        mesh=_mesh(),
        scratch_types=[
            pltpu.VMEM((128,), jnp.int32),
            pltpu.VMEM((128,), jnp.int32),
            pltpu.VMEM((128, _D), jnp.float32),
            pltpu.VMEM((16,), jnp.int32),
            pltpu.VMEM_SHARED((nh + _GARB, _D), jnp.float32),
            pltpu.SemaphoreType.DMA,
        ],
        compiler_params=pltpu.CompilerParams(needs_layout_passes=False),
        name=f"sc_hop_b{block}",
    )


def _unpool_body(feat_hbm, pa_hbm, pb_hbm,
                 out_hbm,
                 av, bv, ra, rb, ro, sem,
                 *, n_old, pp, np_next, copy_ch):
    c = lax.axis_index("c")
    s = lax.axis_index("s")
    wid = s * 2 + c
    ncopy = n_old // copy_ch

    def copy_grp(it, _):
        gi = it * 32 + wid

        @pl.when(gi < ncopy)
        def _():
            pltpu.sync_copy(feat_hbm.at[pl.ds(gi * copy_ch, copy_ch)],
                            ra.at[pl.ds(0, copy_ch)])
            pltpu.sync_copy(ra.at[pl.ds(0, copy_ch)],
                            out_hbm.at[pl.ds(gi * copy_ch, copy_ch)])
        return 0

    lax.fori_loop(0, (ncopy + 31) // 32, copy_grp, 0)

    gp = pp // 128

    def pool_grp(it, _):
        gi = it * 32 + wid

        @pl.when(gi < gp)
        def _():
            pltpu.sync_copy(pa_hbm.at[gi], av)
            pltpu.sync_copy(pb_hbm.at[gi], bv)
            pltpu.async_copy(feat_hbm.at[av], ra, sem).wait()
            pltpu.async_copy(feat_hbm.at[bv], rb, sem).wait()

            def row_body(r, _):
                for kk in range(_D // 16):
                    sl = pl.ds(kk * 16, 16)
                    ro[r, sl] = 0.5 * (ra[r, sl] + rb[r, sl])
                return 0

            lax.fori_loop(0, 128, row_body, 0)
            pltpu.sync_copy(ro, out_hbm.at[pl.ds(n_old + gi * 128, 128)])
        return 0

    lax.fori_loop(0, (gp + 31) // 32, pool_grp, 0)

    ztail = np_next - (n_old + pp)

    @pl.when((c == 0) & (s == 0))
    def _zero_tail():
        def zfill(i, _):
            for kk in range(_D // 16):
                ro[i, pl.ds(kk * 16, 16)] = jnp.zeros((16,), jnp.float32)
            return 0
        lax.fori_loop(0, 128, zfill, 0)
        full, rem = ztail // 128, ztail % 128
        for kk in range(full):
            pltpu.sync_copy(ro, out_hbm.at[pl.ds(n_old + pp + kk * 128, 128)])
        if rem:
            pltpu.sync_copy(ro.at[pl.ds(0, rem)],
                            out_hbm.at[pl.ds(n_old + pp + full * 128, rem)])


@functools.lru_cache(maxsize=None)
def _unpool_fn(n_old, pp, np_next, copy_ch):
    return pl.kernel(
        functools.partial(_unpool_body, n_old=n_old, pp=pp,
                          np_next=np_next, copy_ch=copy_ch),
        out_type=jax.ShapeDtypeStruct((np_next, _D), jnp.float32),
        mesh=_mesh(),
        scratch_types=[
            pltpu.VMEM((128,), jnp.int32),
            pltpu.VMEM((128,), jnp.int32),
            pltpu.VMEM((128, _D), jnp.float32),
            pltpu.VMEM((128, _D), jnp.float32),
            pltpu.VMEM((128, _D), jnp.float32),
            pltpu.SemaphoreType.DMA,
        ],
        compiler_params=pltpu.CompilerParams(needs_layout_passes=False),
        name=f"sc_unpool_{n_old}",
    )


def _mm_body(x_ref, g1_ref, g2_ref, norm_ref, w_ref, b_ref, out_ref, *, relu):
    nrm = norm_ref[...]
    h0 = x_ref[...]
    h1 = g1_ref[...] * nrm
    h2 = g2_ref[...] * nrm
    cat = jnp.concatenate([h0, h1, h2], axis=1)
    acc = jnp.dot(cat, w_ref[...], preferred_element_type=jnp.float32)
    acc = acc + b_ref[...]
    if relu:
        acc = jnp.maximum(acc, 0.0)
    out_ref[...] = acc


def _mm_layer(x, g1, g2, norm2d, w, b, *, relu):
    np_rows = x.shape[0]
    tr = 512
    grid = (np_rows // tr,)
    bs_rows = pl.BlockSpec((tr, _D), lambda i: (i, 0))
    bs_norm = pl.BlockSpec((tr, 1), lambda i: (i, 0))
    bs_w = pl.BlockSpec((3 * _D, _D), lambda i: (0, 0))
    bs_b = pl.BlockSpec((1, _D), lambda i: (0, 0))
    fn = pl.pallas_call(
        functools.partial(_mm_body, relu=relu),
        grid=grid,
        in_specs=[bs_rows, bs_rows, bs_rows, bs_norm, bs_w, bs_b],
        out_specs=bs_rows,
        out_shape=jax.ShapeDtypeStruct((np_rows, _D), jnp.float32),
        compiler_params=pltpu.CompilerParams(
            dimension_semantics=("arbitrary",)),
    )
    return fn(x, g1, g2, norm2d, w, b.reshape(1, _D))


def _pad_rows(x, npad):
    return jnp.pad(x, ((0, npad - x.shape[0]), (0, 0)))


def _prep_edges(ei, block):
    twoe = 2 * ei.shape[1]
    src = jnp.concatenate([ei[0], ei[1]])
    dst = jnp.concatenate([ei[1], ei[0]])
    pad = _EP[block] - twoe
    src = jnp.concatenate([src, jnp.zeros((pad,), jnp.int32)])
    dst = jnp.concatenate([dst, jnp.full((pad,), _NP[block] + 7, jnp.int32)])
    return src, dst


def _block(h, Ws, bs, srcp, dstp, counts, norm2d, block):
    eltwise = (2, 4, 6, 8, 10, 12)
    hop = _hop_fn(block)
    hidden = []
    t1 = h * norm2d
    for i in range(14):
        relu = i < 13
        g1 = hop(t1, srcp, dstp, counts)
        t2 = (g1 * norm2d) * norm2d
        g2 = hop(t2, srcp, dstp, counts)
        out = _mm_layer(h, g1, g2, norm2d, Ws[i], bs[i], relu=relu)
        hidden.append(out)
        h = out
        if i in eltwise:
            h = 0.5 * (hidden[-2] + h)
        t1 = h * norm2d
    return h


def kernel(features, edge_index0, pool_idx0, edge_index1, pool_idx1,
           edge_index2, W, b):
    outs = []
    h = _pad_rows(features, _NP[0])
    pool = (pool_idx0, pool_idx1)
    npool = (_N0, _N1)
    ei = (edge_index0, edge_index1, edge_index2)
    for blk in range(3):
        nh = _NH[blk]
        src, dst = _prep_edges(ei[blk], blk)
        srcp, dstp, counts, deg = _partition_fn(blk)(src, dst)
        norm = jnp.power(jnp.clip(deg, 1.0, None), -0.5)
        norm2d = norm[:, None]
        h = _block(h, W[14 * blk:14 * blk + 14], b[14 * blk:14 * blk + 14],
                   srcp, dstp, counts, norm2d, blk)
        outs.append(h)
        if blk < 2:
            p = pool[blk]
            pp = ((p.shape[0] + 127) // 128) * 128
            padn = pp - p.shape[0]
            pa = jnp.concatenate([p[:, 0], jnp.broadcast_to(p[-1, 0], (padn,))])
            pb = jnp.concatenate([p[:, 1], jnp.broadcast_to(p[-1, 1], (padn,))])
            pa = pa.reshape(pp // 128, 128)
            pb = pb.reshape(pp // 128, 128)
            h = _unpool_fn(npool[blk], pp, _NP[blk + 1], 40)(h, pa, pb)
            outs.append(h)

    out1, h1, out2, h2, out3 = outs
    return (out1[:_N0], out2[:_N1], out3[:_N2], h1[:_N1], h2[:_N2])


# speedup vs baseline: 7.5134x; 1.5050x over previous
import functools

import jax
import jax.numpy as jnp
from jax import lax
from jax.experimental import pallas as pl
from jax.experimental.pallas import tpu as pltpu
from jax.experimental.pallas import tpu_sc as plsc

_N0, _N1, _N2 = 10000, 15000, 22500
_D = 128

_NP = {0: 10240, 1: 15360, 2: 23040}
_NH = {b: _NP[b] // 2 for b in _NP}
_CH = 1024
_EP = {0: 327680, 1: 491520, 2: 737280}
_SCAN = {b: _EP[b] // 16 for b in _EP}
_G = {b: _SCAN[b] // 128 for b in _SCAN}
_GARB = 32


def _mesh():
    return plsc.VectorSubcoreMesh(core_axis_name="c", subcore_axis_name="s")


def _partition_body(src_hbm, dst_hbm,
                    combp_hbm, counts_hbm, deg_hbm,
                    sv, dv, combp_v, onev, cntv, z1, deg_sp, sem,
                    *, nh, scan, g):
    c = lax.axis_index("c")
    s = lax.axis_index("s")
    lo = c * nh
    iota = lax.iota(jnp.int32, 16)

    @pl.when(s == 0)
    def _zero_deg():
        def zfill(i, _):
            z1[pl.ds(i * 16, 16)] = jnp.zeros((16,), jnp.float32)
            return 0
        lax.fori_loop(0, 64, zfill, 0)
        full, rem = (nh + _GARB) // 1024, (nh + _GARB) % 1024
        for kk in range(full):
            pltpu.sync_copy(z1, deg_sp.at[pl.ds(kk * 1024, 1024)])
        if rem:
            pltpu.sync_copy(z1.at[pl.ds(0, rem)],
                            deg_sp.at[pl.ds(full * 1024, rem)])

    for i in range(8):
        onev[pl.ds(i * 16, 16)] = jnp.ones((16,), jnp.float32)

    base = s * scan

    def chunk_body(k, off):
        pltpu.sync_copy(src_hbm.at[pl.ds(base + k * _CH, _CH)], sv)
        pltpu.sync_copy(dst_hbm.at[pl.ds(base + k * _CH, _CH)], dv)

        def vec_body(j, off):
            s16 = sv[pl.ds(j * 16, 16)]
            d16 = dv[pl.ds(j * 16, 16)]
            m = (d16 >= lo) & (d16 < lo + nh)
            inc = jnp.where(m, 1, 0)
            pos = off + plsc.cumsum(inc) - 1
            row = lax.shift_right_logical(pos, 7)
            col = jnp.bitwise_and(pos, 127)
            zz = jnp.zeros((16,), jnp.int32)
            plsc.store_scatter(combp_v, [row, zz, col], s16, mask=m)
            plsc.store_scatter(combp_v, [row, zz + 1, col], d16 - lo, mask=m)
            return off + jnp.max(plsc.all_reduce_population_count(m))

        return lax.fori_loop(0, _CH // 16, vec_body, off)

    off = lax.fori_loop(0, scan // _CH, chunk_body, 0)

    target = ((off + 127) // 128) * 128

    def pad_cond(o):
        return o < target

    def pad_body(o):
        lanes = o + iota
        m = lanes < target
        row = lax.shift_right_logical(lanes, 7)
        col = jnp.bitwise_and(lanes, 127)
        zz = jnp.zeros((16,), jnp.int32)
        plsc.store_scatter(combp_v, [row, zz, col], iota + s * 16, mask=m)
        plsc.store_scatter(combp_v, [row, zz + 1, col],
                           jnp.full((16,), nh + s, jnp.int32), mask=m)
        return o + jnp.max(plsc.all_reduce_population_count(m))

    target = lax.while_loop(pad_cond, pad_body, off)

    cntv[...] = jnp.full((16,), target, jnp.int32)
    pltpu.sync_copy(cntv, counts_hbm.at[c, s])
    pltpu.sync_copy(combp_v, combp_hbm.at[c, s])

    plsc.subcore_barrier()

    def deg_body(gi, _):
        pltpu.sync_copy(onev, deg_sp.at[combp_v.at[gi, 1]], add=True)
        return 0

    lax.fori_loop(0, target // 128, deg_body, 0)
    plsc.subcore_barrier()

    @pl.when(s == 0)
    def _drain_deg():
        full, rem = nh // 1024, nh % 1024
        for kk in range(full):
            pltpu.sync_copy(deg_sp.at[pl.ds(kk * 1024, 1024)], z1)
            pltpu.sync_copy(z1, deg_hbm.at[pl.ds(c * nh + kk * 1024, 1024)])
        if rem:
            pltpu.sync_copy(deg_sp.at[pl.ds(full * 1024, rem)],
                            z1.at[pl.ds(0, rem)])
            pltpu.sync_copy(z1.at[pl.ds(0, rem)],
                            deg_hbm.at[pl.ds(c * nh + full * 1024, rem)])


@functools.lru_cache(maxsize=None)
def _partition_fn(block):
    nh, scan, g = _NH[block], _SCAN[block], _G[block]
    npad = _NP[block]
    return pl.kernel(
        functools.partial(_partition_body, nh=nh, scan=scan, g=g),
        out_type=[
            jax.ShapeDtypeStruct((2, 16, g, 2, 128), jnp.int32),
            jax.ShapeDtypeStruct((2, 16, 16), jnp.int32),
            jax.ShapeDtypeStruct((npad,), jnp.float32),
        ],
        mesh=_mesh(),
        scratch_types=[
            pltpu.VMEM((_CH,), jnp.int32),
            pltpu.VMEM((_CH,), jnp.int32),
            pltpu.VMEM((g, 2, 128), jnp.int32),
            pltpu.VMEM((128,), jnp.float32),
            pltpu.VMEM((16,), jnp.int32),
            pltpu.VMEM((1024,), jnp.float32),
            pltpu.VMEM_SHARED((nh + _GARB,), jnp.float32),
            pltpu.SemaphoreType.DMA,
        ],
        compiler_params=pltpu.CompilerParams(needs_layout_passes=False),
        name=f"sc_partition_b{block}",
    )


def _hop_body(t_hbm, combp_hbm, counts_hbm,
              g_hbm,
              comb0, comb1,
              rows0, rows1,
              cntv,
              isem0, isem1,
              gsem0, gsem1,
              ssem0, ssem1,
              acc_sp,
              *, nh):
    c = lax.axis_index("c")
    s = lax.axis_index("s")
    zr = (nh + _GARB) // 16
    dr = nh // 16
    combs = (comb0, comb1)
    rows = (rows0, rows1)
    isems = (isem0, isem1)
    gsems = (gsem0, gsem1)
    ssems = (ssem0, ssem1)

    def zfill(i, _):
        for kk in range(_D // 16):
            rows0[i, pl.ds(kk * 16, 16)] = jnp.zeros((16,), jnp.float32)
        return 0
    lax.fori_loop(0, 128, zfill, 0)
    zfull, zrem = zr // 128, zr % 128
    for kk in range(zfull):
        pltpu.sync_copy(rows0, acc_sp.at[pl.ds(s * zr + kk * 128, 128)])
    if zrem:
        pltpu.sync_copy(rows0.at[pl.ds(0, zrem)],
                        acc_sp.at[pl.ds(s * zr + zfull * 128, zrem)])
    plsc.subcore_barrier()

    pltpu.sync_copy(counts_hbm.at[c, s], cntv)
    cnt = jnp.max(cntv[...])
    n = cnt // 128

    for b in range(2):
        @pl.when(b < n)
        def _(b=b):
            pltpu.async_copy(combp_hbm.at[c, s, b], combs[b], isems[b]).wait()
            pltpu.async_copy(t_hbm.at[combs[b].at[0]], rows[b], gsems[b])

    def body(it, _):
        base = it * 2
        for b in range(2):
            i = base + b

            @pl.when(i < n)
            def _(b=b, i=i):
                pltpu.make_async_copy(
                    t_hbm.at[combs[b].at[0]], rows[b], gsems[b]).wait()
                pltpu.async_copy(rows[b], acc_sp.at[combs[b].at[1]],
                                 ssems[b], add=True)
        for b in range(2):
            j = base + b + 2

            @pl.when(j < n)
            def _(b=b, j=j):
                pltpu.make_async_copy(rows[b], acc_sp.at[combs[b].at[1]],
                                      ssems[b]).wait()
                pltpu.async_copy(combp_hbm.at[c, s, j], combs[b], isems[b])
        for b in range(2):
            j = base + b + 2

            @pl.when(j < n)
            def _(b=b, j=j):
                pltpu.make_async_copy(
                    combp_hbm.at[c, s, j], combs[b], isems[b]).wait()
                pltpu.async_copy(t_hbm.at[combs[b].at[0]], rows[b], gsems[b])
        return 0

    lax.fori_loop(0, (n + 1) // 2, body, 0)

    for b in range(2):
        @pl.when(b < n)
        def _(b=b):
            pltpu.make_async_copy(rows[b], acc_sp.at[combs[b].at[1]],
                                  ssems[b]).wait()

    plsc.subcore_barrier()

    dfull, drem = dr // 128, dr % 128
    for kk in range(dfull):
        b = kk % 2
        pltpu.sync_copy(acc_sp.at[pl.ds(s * dr + kk * 128, 128)], rows[b])
        pltpu.sync_copy(rows[b],
                        g_hbm.at[pl.ds(c * nh + s * dr + kk * 128, 128)])
    if drem:
        pltpu.sync_copy(acc_sp.at[pl.ds(s * dr + dfull * 128, drem)],
                        rows0.at[pl.ds(0, drem)])
        pltpu.sync_copy(rows0.at[pl.ds(0, drem)],
                        g_hbm.at[pl.ds(c * nh + s * dr + dfull * 128, drem)])


@functools.lru_cache(maxsize=None)
def _hop_fn(block):
    nh, g = _NH[block], _G[block]
    npad = _NP[block]
    return pl.kernel(
        functools.partial(_hop_body, nh=nh),
        out_type=jax.ShapeDtypeStruct((npad, _D), jnp.float32),
        mesh=_mesh(),
        scratch_types=(
            [pltpu.VMEM((2, 128), jnp.int32) for _ in range(2)] +
            [pltpu.VMEM((128, _D), jnp.float32) for _ in range(2)] +
            [pltpu.VMEM((16,), jnp.int32)] +
            [pltpu.SemaphoreType.DMA for _ in range(6)] +
            [pltpu.VMEM_SHARED((nh + _GARB, _D), jnp.float32)]
        ),
        compiler_params=pltpu.CompilerParams(needs_layout_passes=False),
        name=f"sc_hop_b{block}",
    )


def _unpool_body(feat_hbm, pa_hbm, pb_hbm,
                 out_hbm,
                 av, bv, ra, rb, ro, sem,
                 *, n_old, pp, np_next, copy_ch):
    c = lax.axis_index("c")
    s = lax.axis_index("s")
    wid = s * 2 + c
    ncopy = n_old // copy_ch

    def copy_grp(it, _):
        gi = it * 32 + wid

        @pl.when(gi < ncopy)
        def _():
            pltpu.sync_copy(feat_hbm.at[pl.ds(gi * copy_ch, copy_ch)],
                            ra.at[pl.ds(0, copy_ch)])
            pltpu.sync_copy(ra.at[pl.ds(0, copy_ch)],
                            out_hbm.at[pl.ds(gi * copy_ch, copy_ch)])
        return 0

    lax.fori_loop(0, (ncopy + 31) // 32, copy_grp, 0)

    gp = pp // 128

    def pool_grp(it, _):
        gi = it * 32 + wid

        @pl.when(gi < gp)
        def _():
            pltpu.sync_copy(pa_hbm.at[gi], av)
            pltpu.sync_copy(pb_hbm.at[gi], bv)
            pltpu.async_copy(feat_hbm.at[av], ra, sem).wait()
            pltpu.async_copy(feat_hbm.at[bv], rb, sem).wait()

            def row_body(r, _):
                for kk in range(_D // 16):
                    sl = pl.ds(kk * 16, 16)
                    ro[r, sl] = 0.5 * (ra[r, sl] + rb[r, sl])
                return 0

            lax.fori_loop(0, 128, row_body, 0)
            pltpu.sync_copy(ro, out_hbm.at[pl.ds(n_old + gi * 128, 128)])
        return 0

    lax.fori_loop(0, (gp + 31) // 32, pool_grp, 0)

    ztail = np_next - (n_old + pp)

    @pl.when((c == 0) & (s == 0))
    def _zero_tail():
        def zfill(i, _):
            for kk in range(_D // 16):
                ro[i, pl.ds(kk * 16, 16)] = jnp.zeros((16,), jnp.float32)
            return 0
        lax.fori_loop(0, 128, zfill, 0)
        full, rem = ztail // 128, ztail % 128
        for kk in range(full):
            pltpu.sync_copy(ro, out_hbm.at[pl.ds(n_old + pp + kk * 128, 128)])
        if rem:
            pltpu.sync_copy(ro.at[pl.ds(0, rem)],
                            out_hbm.at[pl.ds(n_old + pp + full * 128, rem)])


@functools.lru_cache(maxsize=None)
def _unpool_fn(n_old, pp, np_next, copy_ch):
    return pl.kernel(
        functools.partial(_unpool_body, n_old=n_old, pp=pp,
                          np_next=np_next, copy_ch=copy_ch),
        out_type=jax.ShapeDtypeStruct((np_next, _D), jnp.float32),
        mesh=_mesh(),
        scratch_types=[
            pltpu.VMEM((128,), jnp.int32),
            pltpu.VMEM((128,), jnp.int32),
            pltpu.VMEM((128, _D), jnp.float32),
            pltpu.VMEM((128, _D), jnp.float32),
            pltpu.VMEM((128, _D), jnp.float32),
            pltpu.SemaphoreType.DMA,
        ],
        compiler_params=pltpu.CompilerParams(needs_layout_passes=False),
        name=f"sc_unpool_{n_old}",
    )


def _mm_body(x_ref, g1_ref, g2_ref, norm_ref, w_ref, b_ref, out_ref, *, relu):
    nrm = norm_ref[...]
    h0 = x_ref[...]
    h1 = g1_ref[...] * nrm
    h2 = g2_ref[...] * nrm
    cat = jnp.concatenate([h0, h1, h2], axis=1)
    acc = jnp.dot(cat, w_ref[...], preferred_element_type=jnp.float32)
    acc = acc + b_ref[...]
    if relu:
        acc = jnp.maximum(acc, 0.0)
    out_ref[...] = acc


def _mm_layer(x, g1, g2, norm2d, w, b, *, relu):
    np_rows = x.shape[0]
    tr = 512
    grid = (np_rows // tr,)
    bs_rows = pl.BlockSpec((tr, _D), lambda i: (i, 0))
    bs_norm = pl.BlockSpec((tr, 1), lambda i: (i, 0))
    bs_w = pl.BlockSpec((3 * _D, _D), lambda i: (0, 0))
    bs_b = pl.BlockSpec((1, _D), lambda i: (0, 0))
    fn = pl.pallas_call(
        functools.partial(_mm_body, relu=relu),
        grid=grid,
        in_specs=[bs_rows, bs_rows, bs_rows, bs_norm, bs_w, bs_b],
        out_specs=bs_rows,
        out_shape=jax.ShapeDtypeStruct((np_rows, _D), jnp.float32),
        compiler_params=pltpu.CompilerParams(
            dimension_semantics=("arbitrary",)),
    )
    return fn(x, g1, g2, norm2d, w, b.reshape(1, _D))


def _pad_rows(x, npad):
    return jnp.pad(x, ((0, npad - x.shape[0]), (0, 0)))


def _prep_edges(ei, block):
    twoe = 2 * ei.shape[1]
    src = jnp.concatenate([ei[0], ei[1]])
    dst = jnp.concatenate([ei[1], ei[0]])
    pad = _EP[block] - twoe
    src = jnp.concatenate([src, jnp.zeros((pad,), jnp.int32)])
    dst = jnp.concatenate([dst, jnp.full((pad,), _NP[block] + 7, jnp.int32)])
    return src, dst


def _block(h, Ws, bs, combp, counts, norm2d, block):
    eltwise = (2, 4, 6, 8, 10, 12)
    hop = _hop_fn(block)
    hidden = []
    t1 = h * norm2d
    for i in range(14):
        relu = i < 13
        g1 = hop(t1, combp, counts)
        t2 = (g1 * norm2d) * norm2d
        g2 = hop(t2, combp, counts)
        out = _mm_layer(h, g1, g2, norm2d, Ws[i], bs[i], relu=relu)
        hidden.append(out)
        h = out
        if i in eltwise:
            h = 0.5 * (hidden[-2] + h)
        t1 = h * norm2d
    return h


def kernel(features, edge_index0, pool_idx0, edge_index1, pool_idx1,
           edge_index2, W, b):
    outs = []
    h = _pad_rows(features, _NP[0])
    pool = (pool_idx0, pool_idx1)
    npool = (_N0, _N1)
    ei = (edge_index0, edge_index1, edge_index2)
    for blk in range(3):
        nh = _NH[blk]
        src, dst = _prep_edges(ei[blk], blk)
        combp, counts, deg = _partition_fn(blk)(src, dst)
        norm = jnp.power(jnp.clip(deg, 1.0, None), -0.5)
        norm2d = norm[:, None]
        h = _block(h, W[14 * blk:14 * blk + 14], b[14 * blk:14 * blk + 14],
                   combp, counts, norm2d, blk)
        outs.append(h)
        if blk < 2:
            p = pool[blk]
            pp = ((p.shape[0] + 127) // 128) * 128
            padn = pp - p.shape[0]
            pa = jnp.concatenate([p[:, 0], jnp.broadcast_to(p[-1, 0], (padn,))])
            pb = jnp.concatenate([p[:, 1], jnp.broadcast_to(p[-1, 1], (padn,))])
            pa = pa.reshape(pp // 128, 128)
            pb = pb.reshape(pp // 128, 128)
            h = _unpool_fn(npool[blk], pp, _NP[blk + 1], 40)(h, pa, pb)
            outs.append(h)

    out1, h1, out2, h2, out3 = outs
    return (out1[:_N0], out2[:_N1], out3[:_N2], h1[:_N1], h2[:_N2])


# speedup vs baseline: 8.1437x; 1.0839x over previous
import functools

import jax
import jax.numpy as jnp
from jax import lax
from jax.experimental import pallas as pl
from jax.experimental.pallas import tpu as pltpu
from jax.experimental.pallas import tpu_sc as plsc

_N0, _N1, _N2 = 10000, 15000, 22500
_D = 128

_NP = {0: 10240, 1: 15360, 2: 23040}
_NH = {b: _NP[b] // 2 for b in _NP}
_CH = 1024
_EP = {0: 327680, 1: 491520, 2: 737280}
_SCAN = {b: _EP[b] // 16 for b in _EP}
_G = {b: _SCAN[b] // 128 for b in _SCAN}
_GARB = 32


def _mesh():
    return plsc.VectorSubcoreMesh(core_axis_name="c", subcore_axis_name="s")


def _partition_body(src_hbm, dst_hbm,
                    combp_hbm, counts_hbm, deg_hbm,
                    sv, dv, combp_v, onev, cntv, z1, deg_sp, sem,
                    *, nh, scan, g):
    c = lax.axis_index("c")
    s = lax.axis_index("s")
    lo = c * nh
    iota = lax.iota(jnp.int32, 16)

    @pl.when(s == 0)
    def _zero_deg():
        def zfill(i, _):
            z1[pl.ds(i * 16, 16)] = jnp.zeros((16,), jnp.float32)
            return 0
        lax.fori_loop(0, 64, zfill, 0)
        full, rem = (nh + _GARB) // 1024, (nh + _GARB) % 1024
        for kk in range(full):
            pltpu.sync_copy(z1, deg_sp.at[pl.ds(kk * 1024, 1024)])
        if rem:
            pltpu.sync_copy(z1.at[pl.ds(0, rem)],
                            deg_sp.at[pl.ds(full * 1024, rem)])

    for i in range(8):
        onev[pl.ds(i * 16, 16)] = jnp.ones((16,), jnp.float32)

    base = s * scan

    def chunk_body(k, off):
        pltpu.sync_copy(src_hbm.at[pl.ds(base + k * _CH, _CH)], sv)
        pltpu.sync_copy(dst_hbm.at[pl.ds(base + k * _CH, _CH)], dv)

        def vec_body(j, off):
            s16 = sv[pl.ds(j * 16, 16)]
            d16 = dv[pl.ds(j * 16, 16)]
            m = (d16 >= lo) & (d16 < lo + nh)
            inc = jnp.where(m, 1, 0)
            pos = off + plsc.cumsum(inc) - 1
            row = lax.shift_right_logical(pos, 7)
            col = jnp.bitwise_and(pos, 127)
            zz = jnp.zeros((16,), jnp.int32)
            plsc.store_scatter(combp_v, [row, zz, col], s16, mask=m)
            plsc.store_scatter(combp_v, [row, zz + 1, col], d16 - lo, mask=m)
            return off + jnp.max(plsc.all_reduce_population_count(m))

        return lax.fori_loop(0, _CH // 16, vec_body, off)

    off = lax.fori_loop(0, scan // _CH, chunk_body, 0)

    target = ((off + 127) // 128) * 128

    def pad_cond(o):
        return o < target

    def pad_body(o):
        lanes = o + iota
        m = lanes < target
        row = lax.shift_right_logical(lanes, 7)
        col = jnp.bitwise_and(lanes, 127)
        zz = jnp.zeros((16,), jnp.int32)
        plsc.store_scatter(combp_v, [row, zz, col], iota + s * 16, mask=m)
        plsc.store_scatter(combp_v, [row, zz + 1, col],
                           jnp.full((16,), nh + s, jnp.int32), mask=m)
        return o + jnp.max(plsc.all_reduce_population_count(m))

    target = lax.while_loop(pad_cond, pad_body, off)

    cntv[...] = jnp.full((16,), target, jnp.int32)
    pltpu.sync_copy(cntv, counts_hbm.at[c, s])
    pltpu.sync_copy(combp_v, combp_hbm.at[c, s])

    plsc.subcore_barrier()

    def deg_body(gi, _):
        pltpu.sync_copy(onev, deg_sp.at[combp_v.at[gi, 1]], add=True)
        return 0

    lax.fori_loop(0, target // 128, deg_body, 0)
    plsc.subcore_barrier()

    @pl.when(s == 0)
    def _drain_deg():
        full, rem = nh // 1024, nh % 1024
        for kk in range(full):
            pltpu.sync_copy(deg_sp.at[pl.ds(kk * 1024, 1024)], z1)
            pltpu.sync_copy(z1, deg_hbm.at[pl.ds(c * nh + kk * 1024, 1024)])
        if rem:
            pltpu.sync_copy(deg_sp.at[pl.ds(full * 1024, rem)],
                            z1.at[pl.ds(0, rem)])
            pltpu.sync_copy(z1.at[pl.ds(0, rem)],
                            deg_hbm.at[pl.ds(c * nh + full * 1024, rem)])


@functools.lru_cache(maxsize=None)
def _partition_fn(block):
    nh, scan, g = _NH[block], _SCAN[block], _G[block]
    npad = _NP[block]
    return pl.kernel(
        functools.partial(_partition_body, nh=nh, scan=scan, g=g),
        out_type=[
            jax.ShapeDtypeStruct((2, 16, g, 2, 128), jnp.int32),
            jax.ShapeDtypeStruct((2, 16, 16), jnp.int32),
            jax.ShapeDtypeStruct((npad,), jnp.float32),
        ],
        mesh=_mesh(),
        scratch_types=[
            pltpu.VMEM((_CH,), jnp.int32),
            pltpu.VMEM((_CH,), jnp.int32),
            pltpu.VMEM((g, 2, 128), jnp.int32),
            pltpu.VMEM((128,), jnp.float32),
            pltpu.VMEM((16,), jnp.int32),
            pltpu.VMEM((1024,), jnp.float32),
            pltpu.VMEM_SHARED((nh + _GARB,), jnp.float32),
            pltpu.SemaphoreType.DMA,
        ],
        compiler_params=pltpu.CompilerParams(needs_layout_passes=False),
        name=f"sc_partition_b{block}",
    )


def _hop_body(*args, nh, depth):
    (t_hbm, combp_hbm, counts_hbm, g_hbm) = args[:4]
    rest = args[4:]
    combs = rest[0:depth]
    rows = rest[depth:2 * depth]
    cntv = rest[2 * depth]
    isems = rest[2 * depth + 1:2 * depth + 1 + depth]
    gsems = rest[2 * depth + 1 + depth:2 * depth + 1 + 2 * depth]
    ssems = rest[2 * depth + 1 + 2 * depth:2 * depth + 1 + 3 * depth]
    acc_sp = rest[2 * depth + 1 + 3 * depth]
    c = lax.axis_index("c")
    s = lax.axis_index("s")
    zr = (nh + _GARB) // 16
    dr = nh // 16
    rows0 = rows[0]

    def zfill(i, _):
        for kk in range(_D // 16):
            rows0[i, pl.ds(kk * 16, 16)] = jnp.zeros((16,), jnp.float32)
        return 0
    lax.fori_loop(0, 128, zfill, 0)
    zfull, zrem = zr // 128, zr % 128
    for kk in range(zfull):
        pltpu.sync_copy(rows0, acc_sp.at[pl.ds(s * zr + kk * 128, 128)])
    if zrem:
        pltpu.sync_copy(rows0.at[pl.ds(0, zrem)],
                        acc_sp.at[pl.ds(s * zr + zfull * 128, zrem)])
    plsc.subcore_barrier()

    pltpu.sync_copy(counts_hbm.at[c, s], cntv)
    cnt = jnp.max(cntv[...])
    n = cnt // 128

    for b in range(depth):
        @pl.when(b < n)
        def _(b=b):
            pltpu.async_copy(combp_hbm.at[c, s, b], combs[b], isems[b]).wait()
            pltpu.async_copy(t_hbm.at[combs[b].at[0]], rows[b], gsems[b])

    def body(it, _):
        base = it * depth
        for b in range(depth):
            i = base + b

            @pl.when(i < n)
            def _(b=b, i=i):
                pltpu.make_async_copy(
                    t_hbm.at[combs[b].at[0]], rows[b], gsems[b]).wait()
                pltpu.async_copy(rows[b], acc_sp.at[combs[b].at[1]],
                                 ssems[b], add=True)
        for b in range(depth):
            j = base + b + depth

            @pl.when(j < n)
            def _(b=b, j=j):
                pltpu.make_async_copy(rows[b], acc_sp.at[combs[b].at[1]],
                                      ssems[b]).wait()
                pltpu.async_copy(combp_hbm.at[c, s, j], combs[b], isems[b])
        for b in range(depth):
            j = base + b + depth

            @pl.when(j < n)
            def _(b=b, j=j):
                pltpu.make_async_copy(
                    combp_hbm.at[c, s, j], combs[b], isems[b]).wait()
                pltpu.async_copy(t_hbm.at[combs[b].at[0]], rows[b], gsems[b])
        return 0

    lax.fori_loop(0, (n + depth - 1) // depth, body, 0)

    for b in range(depth):
        @pl.when(b < n)
        def _(b=b):
            pltpu.make_async_copy(rows[b], acc_sp.at[combs[b].at[1]],
                                  ssems[b]).wait()

    plsc.subcore_barrier()

    dfull, drem = dr // 128, dr % 128
    for kk in range(dfull):
        bb = kk % 2
        pltpu.sync_copy(acc_sp.at[pl.ds(s * dr + kk * 128, 128)], rows[bb])
        pltpu.sync_copy(rows[bb],
                        g_hbm.at[pl.ds(c * nh + s * dr + kk * 128, 128)])
    if drem:
        pltpu.sync_copy(acc_sp.at[pl.ds(s * dr + dfull * 128, drem)],
                        rows0.at[pl.ds(0, drem)])
        pltpu.sync_copy(rows0.at[pl.ds(0, drem)],
                        g_hbm.at[pl.ds(c * nh + s * dr + dfull * 128, drem)])


_DEPTH = {0: 4, 1: 3, 2: 2}


@functools.lru_cache(maxsize=None)
def _hop_fn(block):
    nh, g = _NH[block], _G[block]
    npad = _NP[block]
    depth = _DEPTH[block]
    return pl.kernel(
        functools.partial(_hop_body, nh=nh, depth=depth),
        out_type=jax.ShapeDtypeStruct((npad, _D), jnp.float32),
        mesh=_mesh(),
        scratch_types=(
            [pltpu.VMEM((2, 128), jnp.int32) for _ in range(depth)] +
            [pltpu.VMEM((128, _D), jnp.float32) for _ in range(depth)] +
            [pltpu.VMEM((16,), jnp.int32)] +
            [pltpu.SemaphoreType.DMA for _ in range(3 * depth)] +
            [pltpu.VMEM_SHARED((nh + _GARB, _D), jnp.float32)]
        ),
        compiler_params=pltpu.CompilerParams(needs_layout_passes=False),
        name=f"sc_hop_b{block}",
    )


def _unpool_body(feat_hbm, pa_hbm, pb_hbm,
                 out_hbm,
                 av, bv, ra, rb, ro, sem,
                 *, n_old, pp, np_next, copy_ch):
    c = lax.axis_index("c")
    s = lax.axis_index("s")
    wid = s * 2 + c
    ncopy = n_old // copy_ch

    def copy_grp(it, _):
        gi = it * 32 + wid

        @pl.when(gi < ncopy)
        def _():
            pltpu.sync_copy(feat_hbm.at[pl.ds(gi * copy_ch, copy_ch)],
                            ra.at[pl.ds(0, copy_ch)])
            pltpu.sync_copy(ra.at[pl.ds(0, copy_ch)],
                            out_hbm.at[pl.ds(gi * copy_ch, copy_ch)])
        return 0

    lax.fori_loop(0, (ncopy + 31) // 32, copy_grp, 0)

    gp = pp // 128

    def pool_grp(it, _):
        gi = it * 32 + wid

        @pl.when(gi < gp)
        def _():
            pltpu.sync_copy(pa_hbm.at[gi], av)
            pltpu.sync_copy(pb_hbm.at[gi], bv)
            pltpu.async_copy(feat_hbm.at[av], ra, sem).wait()
            pltpu.async_copy(feat_hbm.at[bv], rb, sem).wait()

            def row_body(r, _):
                for kk in range(_D // 16):
                    sl = pl.ds(kk * 16, 16)
                    ro[r, sl] = 0.5 * (ra[r, sl] + rb[r, sl])
                return 0

            lax.fori_loop(0, 128, row_body, 0)
            pltpu.sync_copy(ro, out_hbm.at[pl.ds(n_old + gi * 128, 128)])
        return 0

    lax.fori_loop(0, (gp + 31) // 32, pool_grp, 0)

    ztail = np_next - (n_old + pp)

    @pl.when((c == 0) & (s == 0))
    def _zero_tail():
        def zfill(i, _):
            for kk in range(_D // 16):
                ro[i, pl.ds(kk * 16, 16)] = jnp.zeros((16,), jnp.float32)
            return 0
        lax.fori_loop(0, 128, zfill, 0)
        full, rem = ztail // 128, ztail % 128
        for kk in range(full):
            pltpu.sync_copy(ro, out_hbm.at[pl.ds(n_old + pp + kk * 128, 128)])
        if rem:
            pltpu.sync_copy(ro.at[pl.ds(0, rem)],
                            out_hbm.at[pl.ds(n_old + pp + full * 128, rem)])


@functools.lru_cache(maxsize=None)
def _unpool_fn(n_old, pp, np_next, copy_ch):
    return pl.kernel(
        functools.partial(_unpool_body, n_old=n_old, pp=pp,
                          np_next=np_next, copy_ch=copy_ch),
        out_type=jax.ShapeDtypeStruct((np_next, _D), jnp.float32),
        mesh=_mesh(),
        scratch_types=[
            pltpu.VMEM((128,), jnp.int32),
            pltpu.VMEM((128,), jnp.int32),
            pltpu.VMEM((128, _D), jnp.float32),
            pltpu.VMEM((128, _D), jnp.float32),
            pltpu.VMEM((128, _D), jnp.float32),
            pltpu.SemaphoreType.DMA,
        ],
        compiler_params=pltpu.CompilerParams(needs_layout_passes=False),
        name=f"sc_unpool_{n_old}",
    )


def _mm_body(x_ref, g1_ref, g2_ref, norm_ref, w_ref, b_ref, out_ref, *, relu):
    nrm = norm_ref[...]
    h0 = x_ref[...]
    h1 = g1_ref[...] * nrm
    h2 = g2_ref[...] * nrm
    cat = jnp.concatenate([h0, h1, h2], axis=1)
    acc = jnp.dot(cat, w_ref[...], preferred_element_type=jnp.float32)
    acc = acc + b_ref[...]
    if relu:
        acc = jnp.maximum(acc, 0.0)
    out_ref[...] = acc


def _mm_layer(x, g1, g2, norm2d, w, b, *, relu):
    np_rows = x.shape[0]
    tr = 512
    grid = (np_rows // tr,)
    bs_rows = pl.BlockSpec((tr, _D), lambda i: (i, 0))
    bs_norm = pl.BlockSpec((tr, 1), lambda i: (i, 0))
    bs_w = pl.BlockSpec((3 * _D, _D), lambda i: (0, 0))
    bs_b = pl.BlockSpec((1, _D), lambda i: (0, 0))
    fn = pl.pallas_call(
        functools.partial(_mm_body, relu=relu),
        grid=grid,
        in_specs=[bs_rows, bs_rows, bs_rows, bs_norm, bs_w, bs_b],
        out_specs=bs_rows,
        out_shape=jax.ShapeDtypeStruct((np_rows, _D), jnp.float32),
        compiler_params=pltpu.CompilerParams(
            dimension_semantics=("arbitrary",)),
    )
    return fn(x, g1, g2, norm2d, w, b.reshape(1, _D))


def _pad_rows(x, npad):
    return jnp.pad(x, ((0, npad - x.shape[0]), (0, 0)))


def _prep_edges(ei, block):
    twoe = 2 * ei.shape[1]
    src = jnp.concatenate([ei[0], ei[1]])
    dst = jnp.concatenate([ei[1], ei[0]])
    pad = _EP[block] - twoe
    src = jnp.concatenate([src, jnp.zeros((pad,), jnp.int32)])
    dst = jnp.concatenate([dst, jnp.full((pad,), _NP[block] + 7, jnp.int32)])
    return src, dst


def _block(h, Ws, bs, combp, counts, norm2d, block):
    eltwise = (2, 4, 6, 8, 10, 12)
    hop = _hop_fn(block)
    hidden = []
    t1 = h * norm2d
    for i in range(14):
        relu = i < 13
        g1 = hop(t1, combp, counts)
        t2 = (g1 * norm2d) * norm2d
        g2 = hop(t2, combp, counts)
        out = _mm_layer(h, g1, g2, norm2d, Ws[i], bs[i], relu=relu)
        hidden.append(out)
        h = out
        if i in eltwise:
            h = 0.5 * (hidden[-2] + h)
        t1 = h * norm2d
    return h


def kernel(features, edge_index0, pool_idx0, edge_index1, pool_idx1,
           edge_index2, W, b):
    outs = []
    h = _pad_rows(features, _NP[0])
    pool = (pool_idx0, pool_idx1)
    npool = (_N0, _N1)
    ei = (edge_index0, edge_index1, edge_index2)
    for blk in range(3):
        nh = _NH[blk]
        src, dst = _prep_edges(ei[blk], blk)
        combp, counts, deg = _partition_fn(blk)(src, dst)
        norm = jnp.power(jnp.clip(deg, 1.0, None), -0.5)
        norm2d = norm[:, None]
        h = _block(h, W[14 * blk:14 * blk + 14], b[14 * blk:14 * blk + 14],
                   combp, counts, norm2d, blk)
        outs.append(h)
        if blk < 2:
            p = pool[blk]
            pp = ((p.shape[0] + 127) // 128) * 128
            padn = pp - p.shape[0]
            pa = jnp.concatenate([p[:, 0], jnp.broadcast_to(p[-1, 0], (padn,))])
            pb = jnp.concatenate([p[:, 1], jnp.broadcast_to(p[-1, 1], (padn,))])
            pa = pa.reshape(pp // 128, 128)
            pb = pb.reshape(pp // 128, 128)
            h = _unpool_fn(npool[blk], pp, _NP[blk + 1], 40)(h, pa, pb)
            outs.append(h)

    out1, h1, out2, h2, out3 = outs
    return (out1[:_N0], out2[:_N1], out3[:_N2], h1[:_N1], h2[:_N2])


# speedup vs baseline: 8.6610x; 1.0635x over previous
import functools

import jax
import jax.numpy as jnp
from jax import lax
from jax.experimental import pallas as pl
from jax.experimental.pallas import tpu as pltpu
from jax.experimental.pallas import tpu_sc as plsc

_N0, _N1, _N2 = 10000, 15000, 22500
_D = 128

_NP = {0: 10240, 1: 15360, 2: 23040}
_NH = {b: _NP[b] // 2 for b in _NP}
_CH = 1024
_EP = {0: 327680, 1: 491520, 2: 737280}
_SCAN = {b: _EP[b] // 16 for b in _EP}
_G = {b: _SCAN[b] // 128 for b in _SCAN}
_GARB = 32
_GSZ = {0: 128, 1: 128, 2: 128}
_NSPLIT = {0: 1, 1: 1, 2: 2}


def _mesh():
    return plsc.VectorSubcoreMesh(core_axis_name="c", subcore_axis_name="s")


def _partition_body(src_hbm, dst_hbm,
                    combp_hbm, counts_hbm, deg_hbm,
                    sv, dv, combp_v, onev, cntv, z1, deg_sp, sem,
                    *, nh, scan, g, gsz, p):
    c = lax.axis_index("c")
    s = lax.axis_index("s")
    lo = (2 * p + c) * nh
    iota = lax.iota(jnp.int32, 16)

    @pl.when(s == 0)
    def _zero_deg():
        def zfill(i, _):
            z1[pl.ds(i * 16, 16)] = jnp.zeros((16,), jnp.float32)
            return 0
        lax.fori_loop(0, 64, zfill, 0)
        full, rem = (nh + _GARB) // 1024, (nh + _GARB) % 1024
        for kk in range(full):
            pltpu.sync_copy(z1, deg_sp.at[pl.ds(kk * 1024, 1024)])
        if rem:
            pltpu.sync_copy(z1.at[pl.ds(0, rem)],
                            deg_sp.at[pl.ds(full * 1024, rem)])

    for i in range(8):
        onev[pl.ds(i * 16, 16)] = jnp.ones((16,), jnp.float32)

    base = s * scan

    def chunk_body(k, off):
        pltpu.sync_copy(src_hbm.at[pl.ds(base + k * _CH, _CH)], sv)
        pltpu.sync_copy(dst_hbm.at[pl.ds(base + k * _CH, _CH)], dv)

        def vec_body(j, off):
            s16 = sv[pl.ds(j * 16, 16)]
            d16 = dv[pl.ds(j * 16, 16)]
            m = (d16 >= lo) & (d16 < lo + nh)
            inc = jnp.where(m, 1, 0)
            pos = off + plsc.cumsum(inc) - 1
            row = lax.shift_right_logical(pos, gsz.bit_length() - 1)
            col = jnp.bitwise_and(pos, gsz - 1)
            zz = jnp.zeros((16,), jnp.int32)
            plsc.store_scatter(combp_v, [row, zz, col], s16, mask=m)
            plsc.store_scatter(combp_v, [row, zz + 1, col], d16 - lo, mask=m)
            return off + jnp.max(plsc.all_reduce_population_count(m))

        return lax.fori_loop(0, _CH // 16, vec_body, off)

    off = lax.fori_loop(0, scan // _CH, chunk_body, 0)

    target = ((off + 127) // 128) * 128

    def pad_cond(o):
        return o < target

    def pad_body(o):
        lanes = o + iota
        m = lanes < target
        row = lax.shift_right_logical(lanes, gsz.bit_length() - 1)
        col = jnp.bitwise_and(lanes, gsz - 1)
        zz = jnp.zeros((16,), jnp.int32)
        plsc.store_scatter(combp_v, [row, zz, col], iota + s * 16, mask=m)
        plsc.store_scatter(combp_v, [row, zz + 1, col],
                           jnp.full((16,), nh + s, jnp.int32), mask=m)
        return o + jnp.max(plsc.all_reduce_population_count(m))

    target = lax.while_loop(pad_cond, pad_body, off)

    cntv[...] = jnp.full((16,), target, jnp.int32)
    pltpu.sync_copy(cntv, counts_hbm.at[c, s])
    pltpu.sync_copy(combp_v, combp_hbm.at[c, s])

    plsc.subcore_barrier()

    def deg_body(gi, _):
        pltpu.sync_copy(onev.at[pl.ds(0, gsz)],
                        deg_sp.at[combp_v.at[gi, 1]], add=True)
        return 0

    lax.fori_loop(0, target // gsz, deg_body, 0)
    plsc.subcore_barrier()

    @pl.when(s == 0)
    def _drain_deg():
        full, rem = nh // 1024, nh % 1024
        for kk in range(full):
            pltpu.sync_copy(deg_sp.at[pl.ds(kk * 1024, 1024)], z1)
            pltpu.sync_copy(z1, deg_hbm.at[pl.ds(c * nh + kk * 1024, 1024)])
        if rem:
            pltpu.sync_copy(deg_sp.at[pl.ds(full * 1024, rem)],
                            z1.at[pl.ds(0, rem)])
            pltpu.sync_copy(z1.at[pl.ds(0, rem)],
                            deg_hbm.at[pl.ds(c * nh + full * 1024, rem)])


@functools.lru_cache(maxsize=None)
def _partition_fn(block, p):
    scan, g = _SCAN[block], _G[block]
    nsplit = _NSPLIT[block]
    nh = _NP[block] // (2 * nsplit)
    gsz = _GSZ[block]
    ng = _SCAN[block] // gsz
    return pl.kernel(
        functools.partial(_partition_body, nh=nh, scan=scan, g=g, gsz=gsz,
                          p=p),
        out_type=[
            jax.ShapeDtypeStruct((2, 16, ng, 2, gsz), jnp.int32),
            jax.ShapeDtypeStruct((2, 16, 16), jnp.int32),
            jax.ShapeDtypeStruct((2 * nh,), jnp.float32),
        ],
        mesh=_mesh(),
        scratch_types=[
            pltpu.VMEM((_CH,), jnp.int32),
            pltpu.VMEM((_CH,), jnp.int32),
            pltpu.VMEM((ng, 2, gsz), jnp.int32),
            pltpu.VMEM((128,), jnp.float32),
            pltpu.VMEM((16,), jnp.int32),
            pltpu.VMEM((1024,), jnp.float32),
            pltpu.VMEM_SHARED((nh + _GARB,), jnp.float32),
            pltpu.SemaphoreType.DMA,
        ],
        compiler_params=pltpu.CompilerParams(needs_layout_passes=False),
        name=f"sc_partition_b{block}",
    )


def _hop_body(*args, nh, depth, gsz):
    (t_hbm, combp_hbm, counts_hbm, g_hbm) = args[:4]
    rest = args[4:]
    combs = rest[0:depth]
    rows = rest[depth:2 * depth]
    cntv = rest[2 * depth]
    isems = rest[2 * depth + 1:2 * depth + 1 + depth]
    gsems = rest[2 * depth + 1 + depth:2 * depth + 1 + 2 * depth]
    ssems = rest[2 * depth + 1 + 2 * depth:2 * depth + 1 + 3 * depth]
    acc_sp = rest[2 * depth + 1 + 3 * depth]
    c = lax.axis_index("c")
    s = lax.axis_index("s")
    zr = (nh + _GARB) // 16
    dr = nh // 16
    rows0 = rows[0]

    def zfill(i, _):
        for kk in range(_D // 16):
            rows0[i, pl.ds(kk * 16, 16)] = jnp.zeros((16,), jnp.float32)
        return 0
    lax.fori_loop(0, gsz, zfill, 0)
    zfull, zrem = zr // gsz, zr % gsz
    for kk in range(zfull):
        pltpu.sync_copy(rows0, acc_sp.at[pl.ds(s * zr + kk * gsz, gsz)])
    if zrem:
        pltpu.sync_copy(rows0.at[pl.ds(0, zrem)],
                        acc_sp.at[pl.ds(s * zr + zfull * gsz, zrem)])
    plsc.subcore_barrier()

    pltpu.sync_copy(counts_hbm.at[c, s], cntv)
    cnt = jnp.max(cntv[...])
    n = cnt // gsz

    for b in range(depth):
        @pl.when(b < n)
        def _(b=b):
            pltpu.async_copy(combp_hbm.at[c, s, b], combs[b], isems[b]).wait()
            pltpu.async_copy(t_hbm.at[combs[b].at[0]], rows[b], gsems[b])

    def body(it, _):
        base = it * depth
        for b in range(depth):
            i = base + b

            @pl.when(i < n)
            def _(b=b, i=i):
                pltpu.make_async_copy(
                    t_hbm.at[combs[b].at[0]], rows[b], gsems[b]).wait()
                pltpu.async_copy(rows[b], acc_sp.at[combs[b].at[1]],
                                 ssems[b], add=True)
        for b in range(depth):
            j = base + b + depth

            @pl.when(j < n)
            def _(b=b, j=j):
                pltpu.make_async_copy(rows[b], acc_sp.at[combs[b].at[1]],
                                      ssems[b]).wait()
                pltpu.async_copy(combp_hbm.at[c, s, j], combs[b], isems[b])
        for b in range(depth):
            j = base + b + depth

            @pl.when(j < n)
            def _(b=b, j=j):
                pltpu.make_async_copy(
                    combp_hbm.at[c, s, j], combs[b], isems[b]).wait()
                pltpu.async_copy(t_hbm.at[combs[b].at[0]], rows[b], gsems[b])
        return 0

    lax.fori_loop(0, (n + depth - 1) // depth, body, 0)

    for b in range(depth):
        @pl.when(b < n)
        def _(b=b):
            pltpu.make_async_copy(rows[b], acc_sp.at[combs[b].at[1]],
                                  ssems[b]).wait()

    plsc.subcore_barrier()

    dfull, drem = dr // gsz, dr % gsz
    for kk in range(dfull):
        bb = kk % 2
        pltpu.sync_copy(acc_sp.at[pl.ds(s * dr + kk * gsz, gsz)], rows[bb])
        pltpu.sync_copy(rows[bb],
                        g_hbm.at[pl.ds(c * nh + s * dr + kk * gsz, gsz)])
    if drem:
        pltpu.sync_copy(acc_sp.at[pl.ds(s * dr + dfull * gsz, drem)],
                        rows0.at[pl.ds(0, drem)])
        pltpu.sync_copy(rows0.at[pl.ds(0, drem)],
                        g_hbm.at[pl.ds(c * nh + s * dr + dfull * gsz, drem)])


_DEPTH = {0: 4, 1: 3, 2: 4}


@functools.lru_cache(maxsize=None)
def _hop_fn(block):
    g = _G[block]
    nsplit = _NSPLIT[block]
    nh = _NP[block] // (2 * nsplit)
    depth = _DEPTH[block]
    gsz = _GSZ[block]
    return pl.kernel(
        functools.partial(_hop_body, nh=nh, depth=depth, gsz=gsz),
        out_type=jax.ShapeDtypeStruct((2 * nh, _D), jnp.float32),
        mesh=_mesh(),
        scratch_types=(
            [pltpu.VMEM((2, gsz), jnp.int32) for _ in range(depth)] +
            [pltpu.VMEM((gsz, _D), jnp.float32) for _ in range(depth)] +
            [pltpu.VMEM((16,), jnp.int32)] +
            [pltpu.SemaphoreType.DMA for _ in range(3 * depth)] +
            [pltpu.VMEM_SHARED((nh + _GARB, _D), jnp.float32)]
        ),
        compiler_params=pltpu.CompilerParams(needs_layout_passes=False),
        name=f"sc_hop_b{block}",
    )


def _unpool_body(feat_hbm, pa_hbm, pb_hbm,
                 out_hbm,
                 av, bv, ra, rb, ro, sem,
                 *, n_old, pp, np_next, copy_ch):
    c = lax.axis_index("c")
    s = lax.axis_index("s")
    wid = s * 2 + c
    ncopy = n_old // copy_ch

    def copy_grp(it, _):
        gi = it * 32 + wid

        @pl.when(gi < ncopy)
        def _():
            pltpu.sync_copy(feat_hbm.at[pl.ds(gi * copy_ch, copy_ch)],
                            ra.at[pl.ds(0, copy_ch)])
            pltpu.sync_copy(ra.at[pl.ds(0, copy_ch)],
                            out_hbm.at[pl.ds(gi * copy_ch, copy_ch)])
        return 0

    lax.fori_loop(0, (ncopy + 31) // 32, copy_grp, 0)

    gp = pp // 128

    def pool_grp(it, _):
        gi = it * 32 + wid

        @pl.when(gi < gp)
        def _():
            pltpu.sync_copy(pa_hbm.at[gi], av)
            pltpu.sync_copy(pb_hbm.at[gi], bv)
            pltpu.async_copy(feat_hbm.at[av], ra, sem).wait()
            pltpu.async_copy(feat_hbm.at[bv], rb, sem).wait()

            def row_body(r, _):
                for kk in range(_D // 16):
                    sl = pl.ds(kk * 16, 16)
                    ro[r, sl] = 0.5 * (ra[r, sl] + rb[r, sl])
                return 0

            lax.fori_loop(0, 128, row_body, 0)
            pltpu.sync_copy(ro, out_hbm.at[pl.ds(n_old + gi * 128, 128)])
        return 0

    lax.fori_loop(0, (gp + 31) // 32, pool_grp, 0)

    ztail = np_next - (n_old + pp)

    @pl.when((c == 0) & (s == 0))
    def _zero_tail():
        def zfill(i, _):
            for kk in range(_D // 16):
                ro[i, pl.ds(kk * 16, 16)] = jnp.zeros((16,), jnp.float32)
            return 0
        lax.fori_loop(0, 128, zfill, 0)
        full, rem = ztail // 128, ztail % 128
        for kk in range(full):
            pltpu.sync_copy(ro, out_hbm.at[pl.ds(n_old + pp + kk * 128, 128)])
        if rem:
            pltpu.sync_copy(ro.at[pl.ds(0, rem)],
                            out_hbm.at[pl.ds(n_old + pp + full * 128, rem)])


@functools.lru_cache(maxsize=None)
def _unpool_fn(n_old, pp, np_next, copy_ch):
    return pl.kernel(
        functools.partial(_unpool_body, n_old=n_old, pp=pp,
                          np_next=np_next, copy_ch=copy_ch),
        out_type=jax.ShapeDtypeStruct((np_next, _D), jnp.float32),
        mesh=_mesh(),
        scratch_types=[
            pltpu.VMEM((128,), jnp.int32),
            pltpu.VMEM((128,), jnp.int32),
            pltpu.VMEM((128, _D), jnp.float32),
            pltpu.VMEM((128, _D), jnp.float32),
            pltpu.VMEM((128, _D), jnp.float32),
            pltpu.SemaphoreType.DMA,
        ],
        compiler_params=pltpu.CompilerParams(needs_layout_passes=False),
        name=f"sc_unpool_{n_old}",
    )


def _mm_body(x_ref, g1_ref, g2_ref, norm_ref, w_ref, b_ref, out_ref, *, relu):
    nrm = norm_ref[...]
    h0 = x_ref[...]
    h1 = g1_ref[...] * nrm
    h2 = g2_ref[...] * nrm
    cat = jnp.concatenate([h0, h1, h2], axis=1)
    acc = jnp.dot(cat, w_ref[...], preferred_element_type=jnp.float32)
    acc = acc + b_ref[...]
    if relu:
        acc = jnp.maximum(acc, 0.0)
    out_ref[...] = acc


def _mm_layer(x, g1, g2, norm2d, w, b, *, relu):
    np_rows = x.shape[0]
    tr = 512
    grid = (np_rows // tr,)
    bs_rows = pl.BlockSpec((tr, _D), lambda i: (i, 0))
    bs_norm = pl.BlockSpec((tr, 1), lambda i: (i, 0))
    bs_w = pl.BlockSpec((3 * _D, _D), lambda i: (0, 0))
    bs_b = pl.BlockSpec((1, _D), lambda i: (0, 0))
    fn = pl.pallas_call(
        functools.partial(_mm_body, relu=relu),
        grid=grid,
        in_specs=[bs_rows, bs_rows, bs_rows, bs_norm, bs_w, bs_b],
        out_specs=bs_rows,
        out_shape=jax.ShapeDtypeStruct((np_rows, _D), jnp.float32),
        compiler_params=pltpu.CompilerParams(
            dimension_semantics=("arbitrary",)),
    )
    return fn(x, g1, g2, norm2d, w, b.reshape(1, _D))


def _pad_rows(x, npad):
    return jnp.pad(x, ((0, npad - x.shape[0]), (0, 0)))


def _prep_edges(ei, block):
    twoe = 2 * ei.shape[1]
    src = jnp.concatenate([ei[0], ei[1]])
    dst = jnp.concatenate([ei[1], ei[0]])
    pad = _EP[block] - twoe
    src = jnp.concatenate([src, jnp.zeros((pad,), jnp.int32)])
    dst = jnp.concatenate([dst, jnp.full((pad,), _NP[block] + 7, jnp.int32)])
    return src, dst


def _block(h, Ws, bs, parts, norm2d, block):
    eltwise = (2, 4, 6, 8, 10, 12)
    hop = _hop_fn(block)

    def seg(t):
        gs = [hop(t, pt[0], pt[1]) for pt in parts]
        return gs[0] if len(gs) == 1 else jnp.concatenate(gs, axis=0)

    hidden = []
    t1 = h * norm2d
    for i in range(14):
        relu = i < 13
        g1 = seg(t1)
        t2 = (g1 * norm2d) * norm2d
        g2 = seg(t2)
        out = _mm_layer(h, g1, g2, norm2d, Ws[i], bs[i], relu=relu)
        hidden.append(out)
        h = out
        if i in eltwise:
            h = 0.5 * (hidden[-2] + h)
        t1 = h * norm2d
    return h


def kernel(features, edge_index0, pool_idx0, edge_index1, pool_idx1,
           edge_index2, W, b):
    outs = []
    h = _pad_rows(features, _NP[0])
    pool = (pool_idx0, pool_idx1)
    npool = (_N0, _N1)
    ei = (edge_index0, edge_index1, edge_index2)
    for blk in range(3):
        nh = _NH[blk]
        src, dst = _prep_edges(ei[blk], blk)
        parts = [_partition_fn(blk, p)(src, dst)
                 for p in range(_NSPLIT[blk])]
        deg = (parts[0][2] if _NSPLIT[blk] == 1
               else jnp.concatenate([pt[2] for pt in parts]))
        norm = jnp.power(jnp.clip(deg, 1.0, None), -0.5)
        norm2d = norm[:, None]
        h = _block(h, W[14 * blk:14 * blk + 14], b[14 * blk:14 * blk + 14],
                   parts, norm2d, blk)
        outs.append(h)
        if blk < 2:
            p = pool[blk]
            pp = ((p.shape[0] + 127) // 128) * 128
            padn = pp - p.shape[0]
            pa = jnp.concatenate([p[:, 0], jnp.broadcast_to(p[-1, 0], (padn,))])
            pb = jnp.concatenate([p[:, 1], jnp.broadcast_to(p[-1, 1], (padn,))])
            pa = pa.reshape(pp // 128, 128)
            pb = pb.reshape(pp // 128, 128)
            h = _unpool_fn(npool[blk], pp, _NP[blk + 1], 40)(h, pa, pb)
            outs.append(h)

    out1, h1, out2, h2, out3 = outs
    return (out1[:_N0], out2[:_N1], out3[:_N2], h1[:_N1], h2[:_N2])
